# baseline (speedup 1.0000x reference)
import jax
import jax.numpy as jnp
from jax import lax
from jax.experimental import pallas as pl
from jax.experimental.pallas import tpu as pltpu
from jax.experimental.pallas import tpu_sc as plsc

_DIM = 128
_B = 4096
_ROWS = 16

_NC = 2
_NS = 16
_NW = _NC * _NS


def _tc_body(keys_ref, queue_ref, out_ref, knt_ref):
    out_ref[...] = queue_ref[...]

    @pl.when(pl.program_id(0) == 0)
    def _normalize():
        k = keys_ref[...]
        norm = jnp.sqrt(jnp.sum(k * k, axis=1, keepdims=True))
        knt_ref[...] = (k / jnp.maximum(norm, 1e-12)).T


def _sc_enqueue_body(knt_hbm, ptr_hbm, q_ref, ptr_vmem, stage_vmem):
    wid = lax.axis_index("s") * _NC + lax.axis_index("c")
    pltpu.sync_copy(ptr_hbm, ptr_vmem)
    p = pl.multiple_of(ptr_vmem[...][0], 128)
    slab = wid // 2
    half = wid % 2
    hw = _B // 2
    pltpu.sync_copy(
        knt_hbm.at[pl.ds(slab * 8, 8), pl.ds(half * hw, hw)], stage_vmem
    )
    pltpu.sync_copy(
        stage_vmem, q_ref.at[pl.ds(slab * 8, 8), pl.ds(p + half * hw, hw)]
    )


def kernel(keys, queue, ptr, filled):
    keys = keys.astype(jnp.float32)
    b, dim = keys.shape
    dim2, kq = queue.shape
    assert dim == _DIM and dim2 == _DIM and b == _B and dim % _ROWS == 0

    qcopy, knt = pl.pallas_call(
        _tc_body,
        grid=(dim // _ROWS,),
        in_specs=[
            pl.BlockSpec((b, dim), lambda r: (0, 0)),
            pl.BlockSpec((_ROWS, kq), lambda r: (r, 0)),
        ],
        out_specs=[
            pl.BlockSpec((_ROWS, kq), lambda r: (r, 0)),
            pl.BlockSpec((dim, b), lambda r: (0, 0)),
        ],
        out_shape=[
            jax.ShapeDtypeStruct((dim, kq), jnp.float32),
            jax.ShapeDtypeStruct((dim, b), jnp.float32),
        ],
    )(keys, queue)

    ptr_vec = jnp.full((16,), ptr, jnp.int32)

    sc_enqueue = pl.kernel(
        _sc_enqueue_body,
        out_type=(),
        mesh=plsc.VectorSubcoreMesh(
            core_axis_name="c", subcore_axis_name="s",
            num_cores=_NC, num_subcores=_NS,
        ),
        scratch_types=[
            pltpu.VMEM((16,), jnp.int32),
            pltpu.VMEM((8, _B // 2), jnp.float32),
        ],
    )

    qref = jax.new_ref(qcopy)
    sc_enqueue(knt, ptr_vec, qref)
    new_queue = jax.freeze(qref)

    new_ptr = jnp.reshape((ptr + b) % kq, (1,)).astype(jnp.int32)
    new_filled = jnp.reshape(jnp.minimum(filled + b, kq), (1,)).astype(jnp.int32)
    return new_queue, new_ptr, new_filled


# speedup vs baseline: 1.0080x; 1.0080x over previous
import jax
import jax.numpy as jnp
from jax import lax
from jax.experimental import pallas as pl
from jax.experimental.pallas import tpu as pltpu
from jax.experimental.pallas import tpu_sc as plsc

_DIM = 128
_B = 4096
_ROWS = 16

_NC = 2
_NS = 16
_NW = _NC * _NS


def _tc_body(keys_ref, queue_ref, out_ref, knt_ref):
    out_ref[...] = queue_ref[...]

    @pl.when(pl.program_id(0) == 0)
    def _normalize():
        k = keys_ref[...]
        norm = jnp.sqrt(jnp.sum(k * k, axis=1, keepdims=True))
        knt_ref[...] = (k / jnp.maximum(norm, 1e-12)).T


def _sc_enqueue_body(knt_hbm, ptr_hbm, q_ref, ptr_vmem, stage_vmem, sem):
    wid = lax.axis_index("s") * _NC + lax.axis_index("c")
    slab = wid // 2
    half = wid % 2
    hw = _B // 2
    stage_in = pltpu.async_copy(
        knt_hbm.at[pl.ds(slab * 8, 8), pl.ds(half * hw, hw)], stage_vmem, sem
    )
    pltpu.sync_copy(ptr_hbm, ptr_vmem)
    p = pl.multiple_of(ptr_vmem[...][0], 128)
    stage_in.wait()
    pltpu.sync_copy(
        stage_vmem, q_ref.at[pl.ds(slab * 8, 8), pl.ds(p + half * hw, hw)]
    )


def kernel(keys, queue, ptr, filled):
    keys = keys.astype(jnp.float32)
    b, dim = keys.shape
    dim2, kq = queue.shape
    assert dim == _DIM and dim2 == _DIM and b == _B and dim % _ROWS == 0

    qcopy, knt = pl.pallas_call(
        _tc_body,
        grid=(dim // _ROWS,),
        in_specs=[
            pl.BlockSpec((b, dim), lambda r: (0, 0)),
            pl.BlockSpec((_ROWS, kq), lambda r: (r, 0)),
        ],
        out_specs=[
            pl.BlockSpec((_ROWS, kq), lambda r: (r, 0)),
            pl.BlockSpec((dim, b), lambda r: (0, 0)),
        ],
        out_shape=[
            jax.ShapeDtypeStruct((dim, kq), jnp.float32),
            jax.ShapeDtypeStruct((dim, b), jnp.float32),
        ],
    )(keys, queue)

    ptr_vec = jnp.full((16,), ptr, jnp.int32)

    sc_enqueue = pl.kernel(
        _sc_enqueue_body,
        out_type=(),
        mesh=plsc.VectorSubcoreMesh(
            core_axis_name="c", subcore_axis_name="s",
            num_cores=_NC, num_subcores=_NS,
        ),
        scratch_types=[
            pltpu.VMEM((16,), jnp.int32),
            pltpu.VMEM((8, _B // 2), jnp.float32),
            pltpu.SemaphoreType.DMA,
        ],
    )

    qref = jax.new_ref(qcopy)
    sc_enqueue(knt, ptr_vec, qref)
    new_queue = jax.freeze(qref)

    new_ptr = jnp.reshape((ptr + b) % kq, (1,)).astype(jnp.int32)
    new_filled = jnp.reshape(jnp.minimum(filled + b, kq), (1,)).astype(jnp.int32)
    return new_queue, new_ptr, new_filled
